# Initial kernel scaffold; baseline (speedup 1.0000x reference)
import jax
import jax.numpy as jnp
from jax.experimental import pallas as pl


def kernel(x, table):
    raise NotImplementedError("write your pallas kernel here")


# speedup vs baseline: 5.7480x; 5.7480x over previous
import functools

import jax
import jax.numpy as jnp
from jax import lax
from jax.experimental import pallas as pl
from jax.experimental.pallas import tpu as pltpu
from jax.experimental.pallas import tpu_sc as plsc

_NC = 2
_NS = 16
_NW = _NC * _NS

_B = 1024 * 200
_D = 128
_BPW = _B // _NW
_CHUNK = 128
_NCHUNK = _BPW // _CHUNK


def _build_gather():
    mesh = plsc.VectorSubcoreMesh(core_axis_name="c", subcore_axis_name="s")

    @functools.partial(
        pl.kernel,
        mesh=mesh,
        out_type=jax.ShapeDtypeStruct((_NW, _NCHUNK, _CHUNK, _D), jnp.float32),
        scratch_types=[
            pltpu.VMEM((_NCHUNK, _CHUNK), jnp.int32),
            pltpu.VMEM((_CHUNK, _D), jnp.float32),
            pltpu.SemaphoreType.DMA,
        ],
    )
    def gather_kernel(idx_hbm, table_hbm, out_hbm, idx_v, rows_v, sem):
        wid = lax.axis_index("s") * _NC + lax.axis_index("c")
        pltpu.sync_copy(idx_hbm.at[wid], idx_v)

        def body(i, carry):
            pltpu.async_copy(table_hbm.at[idx_v.at[i]], rows_v, sem).wait()
            pltpu.sync_copy(rows_v, out_hbm.at[wid, i])
            return carry

        lax.fori_loop(0, _NCHUNK, body, 0)

    return gather_kernel


_GATHER = _build_gather()


def kernel(x, table):
    xf = x.reshape(_NW, _NCHUNK, _CHUNK).astype(jnp.int32)
    out = _GATHER(xf, table)
    return out.reshape(x.shape[0], x.shape[1], _D)


# speedup vs baseline: 7.9896x; 1.3900x over previous
import functools

import jax
import jax.numpy as jnp
from jax import lax
from jax.experimental import pallas as pl
from jax.experimental.pallas import tpu as pltpu
from jax.experimental.pallas import tpu_sc as plsc

_NC = 2
_NS = 16
_NW = _NC * _NS

_B = 1024 * 200
_D = 128
_BPW = _B // _NW
_CHUNK = 128
_NCHUNK = _BPW // _CHUNK
_NBUF = 4


def _build_gather():
    mesh = plsc.VectorSubcoreMesh(core_axis_name="c", subcore_axis_name="s")

    @functools.partial(
        pl.kernel,
        mesh=mesh,
        out_type=jax.ShapeDtypeStruct((_NW, _NCHUNK, _CHUNK, _D), jnp.float32),
        scratch_types=[
            pltpu.VMEM((_NCHUNK, _CHUNK), jnp.int32),
            pltpu.VMEM((_NBUF, _CHUNK, _D), jnp.float32),
            pltpu.SemaphoreType.DMA,
            pltpu.SemaphoreType.DMA,
            pltpu.SemaphoreType.DMA,
            pltpu.SemaphoreType.DMA,
            pltpu.SemaphoreType.DMA,
            pltpu.SemaphoreType.DMA,
            pltpu.SemaphoreType.DMA,
            pltpu.SemaphoreType.DMA,
        ],
    )
    def gather_kernel(idx_hbm, table_hbm, out_hbm, idx_v, rows_v,
                      g0, g1, g2, g3, s0, s1, s2, s3):
        sem_g = (g0, g1, g2, g3)
        sem_s = (s0, s1, s2, s3)
        wid = lax.axis_index("s") * _NC + lax.axis_index("c")
        pltpu.sync_copy(idx_hbm.at[wid], idx_v)

        def start_gather(i, b):
            pltpu.make_async_copy(
                table_hbm.at[idx_v.at[i]], rows_v.at[b], sem_g[b]).start()

        def wait_gather(i, b):
            pltpu.make_async_copy(
                table_hbm.at[idx_v.at[i]], rows_v.at[b], sem_g[b]).wait()

        def start_scatter(i, b):
            pltpu.make_async_copy(
                rows_v.at[b], out_hbm.at[wid, i], sem_s[b]).start()

        def wait_scatter(i, b):
            pltpu.make_async_copy(
                rows_v.at[b], out_hbm.at[wid, i], sem_s[b]).wait()

        def step(i, b, wait_s, start_g):
            wait_gather(i, b)
            start_scatter(i, b)
            b2 = (b + 2) % _NBUF
            if wait_s:
                wait_scatter(i, b2)
            if start_g:
                start_gather(i + 2, b2)

        start_gather(0, 0)
        start_gather(1, 1)
        step(0, 0, wait_s=False, start_g=True)
        step(1, 1, wait_s=False, start_g=True)
        step(2, 2, wait_s=True, start_g=True)
        step(3, 3, wait_s=True, start_g=True)

        def group(g, carry):
            i0 = g * _NBUF
            for b in range(_NBUF):
                step(i0 + b, b, wait_s=True, start_g=True)
            return carry

        lax.fori_loop(1, (_NCHUNK - 2) // _NBUF, group, 0)

        step(_NCHUNK - 2, (_NCHUNK - 2) % _NBUF, wait_s=False, start_g=False)
        step(_NCHUNK - 1, (_NCHUNK - 1) % _NBUF, wait_s=False, start_g=False)
        for i in range(_NCHUNK - 4, _NCHUNK):
            wait_scatter(i, i % _NBUF)

    return gather_kernel


_GATHER = _build_gather()


def kernel(x, table):
    xf = x.reshape(_NW, _NCHUNK, _CHUNK).astype(jnp.int32)
    out = _GATHER(xf, table)
    return out.reshape(x.shape[0], x.shape[1], _D)


# speedup vs baseline: 8.0692x; 1.0100x over previous
import functools

import jax
import jax.numpy as jnp
from jax import lax
from jax.experimental import pallas as pl
from jax.experimental.pallas import tpu as pltpu
from jax.experimental.pallas import tpu_sc as plsc

_NC = 2
_NS = 16
_NW = _NC * _NS

_B = 1024 * 200
_D = 128
_BPW = _B // _NW
_CHUNK = 128
_NCHUNK = _BPW // _CHUNK
_NBUF = 6
_AHEAD = 3


def _build_gather():
    mesh = plsc.VectorSubcoreMesh(core_axis_name="c", subcore_axis_name="s")

    @functools.partial(
        pl.kernel,
        mesh=mesh,
        out_type=jax.ShapeDtypeStruct((_NW, _NCHUNK, _CHUNK, _D), jnp.float32),
        scratch_types=[
            pltpu.VMEM((_NCHUNK, _CHUNK), jnp.int32),
            pltpu.VMEM((_NBUF, _CHUNK, _D), jnp.float32),
        ] + [pltpu.SemaphoreType.DMA] * (2 * _NBUF),
    )
    def gather_kernel(idx_hbm, table_hbm, out_hbm, idx_v, rows_v, *sems):
        sem_g = sems[:_NBUF]
        sem_s = sems[_NBUF:]
        wid = lax.axis_index("s") * _NC + lax.axis_index("c")
        pltpu.sync_copy(idx_hbm.at[wid], idx_v)

        def start_gather(i, b):
            pltpu.make_async_copy(
                table_hbm.at[idx_v.at[i]], rows_v.at[b], sem_g[b]).start()

        def wait_gather(i, b):
            pltpu.make_async_copy(
                table_hbm.at[idx_v.at[i]], rows_v.at[b], sem_g[b]).wait()

        def start_scatter(i, b):
            pltpu.make_async_copy(
                rows_v.at[b], out_hbm.at[wid, i], sem_s[b]).start()

        def wait_scatter(i, b):
            pltpu.make_async_copy(
                rows_v.at[b], out_hbm.at[wid, i], sem_s[b]).wait()

        def step(i, b, wait_s, start_g):
            wait_gather(i, b)
            start_scatter(i, b)
            if start_g:
                b2 = (b + _AHEAD) % _NBUF
                if wait_s:
                    wait_scatter(i, b2)
                start_gather(i + _AHEAD, b2)

        for j in range(_AHEAD):
            start_gather(j, j)
        for i in range(_NBUF):
            step(i, i, wait_s=(i >= _AHEAD), start_g=True)

        n_groups = (_NCHUNK - _AHEAD - _NBUF) // _NBUF

        def group(g, carry):
            i0 = g * _NBUF
            for b in range(_NBUF):
                step(i0 + b, b, wait_s=True, start_g=True)
            return carry

        lax.fori_loop(1, 1 + n_groups, group, 0)

        for i in range(_NBUF * (1 + n_groups), _NCHUNK):
            step(i, i % _NBUF, wait_s=True, start_g=(i + _AHEAD < _NCHUNK))
        for i in range(_NCHUNK - _NBUF, _NCHUNK):
            wait_scatter(i, i % _NBUF)

    return gather_kernel


_GATHER = _build_gather()


def kernel(x, table):
    xf = x.reshape(_NW, _NCHUNK, _CHUNK).astype(jnp.int32)
    out = _GATHER(xf, table)
    return out.reshape(x.shape[0], x.shape[1], _D)
